# Initial kernel scaffold; baseline (speedup 1.0000x reference)
import jax
import jax.numpy as jnp
from jax.experimental import pallas as pl


def kernel(symp_tab, dise_tab, W_dsd_2_1, W_dsd_2_2, W_dsd_1_1, W_dsd_1_2, W_usu_3, W_usu_2_1, W_usu_2_2, W_usu_1, label, dsd_1, dsd_2, usu_1, usu_2, usu_3):
    raise NotImplementedError("write your pallas kernel here")


# speedup vs baseline: 15.9295x; 15.9295x over previous
import functools

import jax
import jax.numpy as jnp
from jax import lax
from jax.experimental import pallas as pl
from jax.experimental.pallas import tpu as pltpu
from jax.experimental.pallas import tpu_sc as plsc

B = 1024
D = 32
NW = 32
F32 = jnp.float32
I32 = jnp.int32

A_GROUPS = 20 * B
B_GROUPS = 25 * B
SEG_CH_GROUPS = 40
A_CHUNKS = A_GROUPS // NW // SEG_CH_GROUPS
B_CHUNKS = B_GROUPS // NW // SEG_CH_GROUPS


def _seg_sum(idx_hbm, tab_hbm, out_hbm, wid, n_chunks, idx_v, rows_v, out_v, sem):
    row_base = wid * (n_chunks * 8)
    grp_base = wid * (n_chunks * SEG_CH_GROUPS)

    def chunk(c, carry):
        rb = row_base + c * 8
        pltpu.sync_copy(idx_hbm.at[pl.ds(rb, 8)], idx_v)
        descs = [
            pltpu.async_copy(tab_hbm.at[idx_v.at[j]],
                             rows_v.at[pl.ds(j * 100, 100)], sem)
            for j in range(8)
        ]
        for dsc in descs:
            dsc.wait()

        def g_body(g, carry2):
            r0 = g * 20
            a0 = rows_v[r0, 0:16]
            a1 = rows_v[r0, 16:32]
            for j in range(1, 20):
                a0 = a0 + rows_v[r0 + j, 0:16]
                a1 = a1 + rows_v[r0 + j, 16:32]
            out_v[g, 0:16] = a0
            out_v[g, 16:32] = a1
            return carry2

        lax.fori_loop(0, SEG_CH_GROUPS, g_body, 0)
        pltpu.sync_copy(out_v, out_hbm.at[pl.ds(grp_base + c * SEG_CH_GROUPS,
                                                SEG_CH_GROUPS)])
        return carry

    lax.fori_loop(0, n_chunks, chunk, 0)


def _gather_rows(idx_hbm, tab_hbm, out_hbm, wid, n_chunks, rows_per_chunk,
                 idx_buf, rows_buf, sem):
    def chunk(c, carry):
        r = wid * n_chunks + c
        pltpu.sync_copy(idx_hbm.at[r], idx_buf)
        pltpu.async_copy(tab_hbm.at[idx_buf], rows_buf, sem).wait()
        pltpu.sync_copy(rows_buf, out_hbm.at[pl.ds(r * rows_per_chunk,
                                                   rows_per_chunk)])
        return carry

    lax.fori_loop(0, n_chunks, chunk, 0)


def _sc_stage(symp_tab, dise_tab, idxA, idxB, idxC, idxD, idxE):
    mesh = plsc.VectorSubcoreMesh(core_axis_name="c", subcore_axis_name="s")

    @functools.partial(
        pl.kernel,
        mesh=mesh,
        compiler_params=pltpu.CompilerParams(use_tc_tiling_on_sc=False),
        out_type=[
            jax.ShapeDtypeStruct((A_GROUPS, D), F32),
            jax.ShapeDtypeStruct((B_GROUPS, D), F32),
            jax.ShapeDtypeStruct((20 * B, D), F32),
            jax.ShapeDtypeStruct((5 * B, D), F32),
            jax.ShapeDtypeStruct((B, D), F32),
        ],
        scratch_types=[
            pltpu.VMEM((8, 100), I32),
            pltpu.VMEM((800, D), F32),
            pltpu.VMEM((SEG_CH_GROUPS, D), F32),
            pltpu.VMEM((128,), I32),
            pltpu.VMEM((80,), I32),
            pltpu.VMEM((32,), I32),
            pltpu.VMEM((128, D), F32),
            pltpu.SemaphoreType.DMA,
        ],
    )
    def sck(symp_hbm, dise_hbm, idxA_hbm, idxB_hbm, idxC_hbm, idxD_hbm,
            idxE_hbm, outA, outB, outC, outD, outE,
            idx_v, rows_v, out_v, idxC_v, idxD_v, idxE_v, rowsg_v, sem):
        wid = lax.axis_index("s") * 2 + lax.axis_index("c")
        _seg_sum(idxA_hbm, dise_hbm, outA, wid, A_CHUNKS, idx_v, rows_v, out_v, sem)
        _seg_sum(idxB_hbm, symp_hbm, outB, wid, B_CHUNKS, idx_v, rows_v, out_v, sem)
        _gather_rows(idxC_hbm, symp_hbm, outC, wid, 5, 128, idxC_v, rowsg_v, sem)
        _gather_rows(idxD_hbm, symp_hbm, outD, wid, 2, 80, idxD_v,
                     rowsg_v.at[pl.ds(0, 80)], sem)
        _gather_rows(idxE_hbm, dise_hbm, outE, wid, 1, 32, idxE_v,
                     rowsg_v.at[pl.ds(0, 32)], sem)

    return sck(symp_tab, dise_tab, idxA, idxB, idxC, idxD, idxE)


def _wfn(cnt):
    w = 1.0 / (cnt + 1e-8)
    return jnp.where(w == 1e8, 0.0, w)


def _leaky(x):
    return jnp.where(x > 0, x, 0.2 * x)


def _tc_body(sumA_ref, sumB_ref, embs_ref, embu1_ref, tgt_ref,
             dsd1_ref, dsd2_ref, usu1_ref, usu2_ref, usu3_ref,
             W21_ref, W22_ref, W11_ref, W12_ref,
             Wu3_ref, Wu21_ref, Wu22_ref, Wu1_ref, out_ref):
    blk = 128
    dot = functools.partial(jnp.dot, preferred_element_type=F32)
    W21, W22 = W21_ref[...], W22_ref[...]
    W11, W12 = W11_ref[...], W12_ref[...]
    Wu3, Wu21, Wu22, Wu1 = Wu3_ref[...], Wu21_ref[...], Wu22_ref[...], Wu1_ref[...]

    cnt2 = jnp.sum((dsd2_ref[...] != 0).astype(F32), axis=-1)
    meand = sumA_ref[...] * _wfn(cnt2)[..., None]
    embs = embs_ref[...]
    X = (meand + embs).reshape(20 * blk, D)
    Y = (meand * embs).reshape(20 * blk, D)
    emb_s_1 = _leaky(dot(X, W21) + dot(Y, W22)).reshape(20, blk, D)
    S1 = jnp.sum(emb_s_1, axis=0)
    cnt1 = jnp.sum((dsd1_ref[...] != 0).astype(F32), axis=-1)
    sbar = S1 * _wfn(cnt1)[:, None]
    tgt = tgt_ref[...]
    emb_dise = _leaky(dot(tgt + sbar, W11) + dot(sbar * tgt, W12))

    cnt3 = jnp.sum((usu3_ref[...] != 0).astype(F32), axis=-1)
    meanu3 = sumB_ref[...] * _wfn(cnt3)[..., None]
    emb_u2 = _leaky(dot(meanu3.reshape(25 * blk, D), Wu3)).reshape(5, 5, blk, D)
    S2 = jnp.sum(emb_u2, axis=1)
    cntu2 = jnp.sum((usu2_ref[...] != 0).astype(F32), axis=-1)
    mbar = S2 * _wfn(cntu2)[..., None]
    embu1 = embu1_ref[...]
    Z = _leaky(dot((embu1 + mbar).reshape(5 * blk, D), Wu21)
               + dot((mbar * embu1).reshape(5 * blk, D), Wu22)).reshape(5, blk, D)
    S3 = jnp.sum(Z, axis=0)
    cntu1 = jnp.sum((usu1_ref[...] != 0).astype(F32), axis=-1)
    ubar = S3 * _wfn(cntu1)[:, None]
    emb_user = _leaky(dot(ubar, Wu1))

    pred = jnp.sum(emb_dise * emb_user, axis=1)
    out_ref[...] = pred.reshape(1, 1, blk)


def _tc_stage(sumA, sumB, embs, embu1, tgt, dsd_1, dsd_2, usu_1, usu_2, usu_3,
              W21, W22, W11, W12, Wu3, Wu21, Wu22, Wu1):
    blk = 128
    g = B // blk
    i3 = lambda i: (0, i, 0)
    i2 = lambda i: (i, 0)
    w2 = lambda i: (0, 0)
    in_specs = [
        pl.BlockSpec((20, blk, D), i3),
        pl.BlockSpec((25, blk, D), i3),
        pl.BlockSpec((20, blk, D), i3),
        pl.BlockSpec((5, blk, D), i3),
        pl.BlockSpec((blk, D), i2),
        pl.BlockSpec((blk, 20), i2),
        pl.BlockSpec((20, blk, 20), i3),
        pl.BlockSpec((blk, 5), i2),
        pl.BlockSpec((5, blk, 5), i3),
        pl.BlockSpec((25, blk, 20), i3),
    ] + [pl.BlockSpec((D, D), w2)] * 8
    out = pl.pallas_call(
        _tc_body,
        grid=(g,),
        in_specs=in_specs,
        out_specs=pl.BlockSpec((1, 1, blk), lambda i: (i, 0, 0)),
        out_shape=jax.ShapeDtypeStruct((g, 1, blk), F32),
    )(sumA, sumB, embs, embu1, tgt, dsd_1, dsd_2, usu_1, usu_2, usu_3,
      W21, W22, W11, W12, Wu3, Wu21, Wu22, Wu1)
    return out.reshape(B)


def kernel(symp_tab, dise_tab, W_dsd_2_1, W_dsd_2_2, W_dsd_1_1, W_dsd_1_2,
           W_usu_3, W_usu_2_1, W_usu_2_2, W_usu_1,
           label, dsd_1, dsd_2, usu_1, usu_2, usu_3):
    dsd_1 = dsd_1.astype(I32)
    dsd_2 = dsd_2.astype(I32)
    usu_1 = usu_1.astype(I32)
    usu_2 = usu_2.astype(I32)
    usu_3 = usu_3.astype(I32)
    label = label.astype(I32)

    idxA = dsd_2.reshape(-1).reshape(A_GROUPS * 20 // 100, 100)
    idxB = usu_3.reshape(-1).reshape(B_GROUPS * 20 // 100, 100)
    idxC = dsd_1.T.reshape(20 * B // 128, 128)
    idxD = usu_1.T.reshape(5 * B // 80, 80)
    idxE = label.reshape(B // 32, 32)

    sumA, sumB, embs, embu1, tgt = _sc_stage(
        symp_tab.astype(F32), dise_tab.astype(F32), idxA, idxB, idxC, idxD, idxE)

    return _tc_stage(
        sumA.reshape(20, B, D), sumB.reshape(25, B, D),
        embs.reshape(20, B, D), embu1.reshape(5, B, D), tgt,
        dsd_1, dsd_2, usu_1, usu_2, usu_3,
        W_dsd_2_1, W_dsd_2_2, W_dsd_1_1, W_dsd_1_2,
        W_usu_3, W_usu_2_1, W_usu_2_2, W_usu_1)


# speedup vs baseline: 17.9446x; 1.1265x over previous
import functools

import jax
import jax.numpy as jnp
from jax import lax
from jax.experimental import pallas as pl
from jax.experimental.pallas import tpu as pltpu
from jax.experimental.pallas import tpu_sc as plsc

B = 1024
D = 32
NW = 32
F32 = jnp.float32
I32 = jnp.int32

A_GROUPS = 20 * B
B_GROUPS = 25 * B
CHG = 32
A_CH = A_GROUPS // NW // CHG
B_CH = B_GROUPS // NW // CHG
NCH = A_CH + B_CH


def _sc_stage(symp_tab, dise_tab, idxA, idxB, idxC, idxD, idxE):
    mesh = plsc.VectorSubcoreMesh(core_axis_name="c", subcore_axis_name="s")

    @functools.partial(
        pl.kernel,
        mesh=mesh,
        compiler_params=pltpu.CompilerParams(use_tc_tiling_on_sc=False),
        out_type=[
            jax.ShapeDtypeStruct((A_GROUPS, D), F32),
            jax.ShapeDtypeStruct((B_GROUPS, D), F32),
            jax.ShapeDtypeStruct((20 * B, D), F32),
            jax.ShapeDtypeStruct((5 * B, D), F32),
            jax.ShapeDtypeStruct((B, D), F32),
        ],
        scratch_types=[
            pltpu.VMEM((100, 128), I32),
            pltpu.VMEM((125, 128), I32),
            pltpu.VMEM((CHG * 20, D), F32),
            pltpu.VMEM((CHG * 20, D), F32),
            pltpu.VMEM((CHG, D), F32),
            pltpu.VMEM((CHG, D), F32),
            pltpu.SemaphoreType.DMA,
            pltpu.SemaphoreType.DMA,
        ],
    )
    def sck(symp_hbm, dise_hbm, idxA_hbm, idxB_hbm, idxC_hbm, idxD_hbm,
            idxE_hbm, outA, outB, outC, outD, outE,
            idxA_v, idxB_v, rows0, rows1, out0, out1, semg, semo):
        wid = lax.axis_index("s") * 2 + lax.axis_index("c")

        def _fire(ci, rows_dst):
            @pl.when(ci < A_CH)
            def _():
                for k in range(5):
                    pltpu.async_copy(dise_hbm.at[idxA_v.at[ci * 5 + k]],
                                     rows_dst.at[pl.ds(k * 128, 128)], semg)

            @pl.when(jnp.logical_not(ci < A_CH))
            def _():
                for k in range(5):
                    pltpu.async_copy(symp_hbm.at[idxB_v.at[(ci - A_CH) * 5 + k]],
                                     rows_dst.at[pl.ds(k * 128, 128)], semg)

        def _drain_gathers():
            for _ in range(5):
                pltpu.make_async_copy(symp_hbm.at[idxB_v.at[0]],
                                      rows0.at[pl.ds(0, 128)], semg).wait()

        def _drain_out():
            pltpu.make_async_copy(out0, outA.at[pl.ds(0, CHG)], semo).wait()

        def _reduce_and_out(c, rows, outv):
            def g_body(g, carry):
                r0 = g * 20
                a0 = rows[r0, 0:16]
                a1 = rows[r0, 16:32]
                for j in range(1, 20):
                    a0 = a0 + rows[r0 + j, 0:16]
                    a1 = a1 + rows[r0 + j, 16:32]
                outv[g, 0:16] = a0
                outv[g, 16:32] = a1
                return carry

            lax.fori_loop(0, CHG, g_body, 0)

            @pl.when(c < A_CH)
            def _():
                pltpu.async_copy(outv, outA.at[pl.ds(wid * (A_CH * CHG) + c * CHG,
                                                     CHG)], semo)

            @pl.when(jnp.logical_not(c < A_CH))
            def _():
                pltpu.async_copy(
                    outv, outB.at[pl.ds(wid * (B_CH * CHG) + (c - A_CH) * CHG,
                                        CHG)], semo)

        pltpu.sync_copy(idxA_hbm.at[pl.ds(wid * 100, 100)], idxA_v)
        pltpu.sync_copy(idxB_hbm.at[pl.ds(wid * 125, 125)], idxB_v)

        _fire(0, rows0)

        def step(c, carry):
            even = jnp.bitwise_and(c, 1) == 0
            has_next = c + 1 < NCH

            @pl.when(jnp.logical_and(has_next, even))
            def _():
                _fire(c + 1, rows1)

            @pl.when(jnp.logical_and(has_next, jnp.logical_not(even)))
            def _():
                _fire(c + 1, rows0)

            _drain_gathers()

            @pl.when(c >= 2)
            def _():
                _drain_out()

            @pl.when(even)
            def _():
                _reduce_and_out(c, rows0, out0)

            @pl.when(jnp.logical_not(even))
            def _():
                _reduce_and_out(c, rows1, out1)

            return carry

        lax.fori_loop(0, NCH, step, 0)
        _drain_out()
        _drain_out()

        pltpu.sync_copy(idxC_hbm.at[pl.ds(wid * 5, 5)], idxA_v.at[pl.ds(0, 5)])
        for k in range(5):
            pltpu.async_copy(symp_hbm.at[idxA_v.at[k]],
                             rows0.at[pl.ds(k * 128, 128)], semg)
        _drain_gathers()
        pltpu.sync_copy(rows0, outC.at[pl.ds(wid * 640, 640)])

        pltpu.sync_copy(idxD_hbm.at[wid], idxA_v.at[0])
        pltpu.async_copy(symp_hbm.at[idxA_v.at[0]],
                         rows0.at[pl.ds(0, 128)], semg).wait()
        pltpu.sync_copy(rows0.at[pl.ds(0, 128)], outD.at[pl.ds(wid * 128, 128)])

        @pl.when(wid < 8)
        def _():
            pltpu.sync_copy(idxD_hbm.at[wid + 32], idxA_v.at[0])
            pltpu.async_copy(symp_hbm.at[idxA_v.at[0]],
                             rows0.at[pl.ds(0, 128)], semg).wait()
            pltpu.sync_copy(rows0.at[pl.ds(0, 128)],
                            outD.at[pl.ds((wid + 32) * 128, 128)])

        @pl.when(wid < 8)
        def _():
            pltpu.sync_copy(idxE_hbm.at[wid], idxA_v.at[0])
            pltpu.async_copy(dise_hbm.at[idxA_v.at[0]],
                             rows0.at[pl.ds(0, 128)], semg).wait()
            pltpu.sync_copy(rows0.at[pl.ds(0, 128)],
                            outE.at[pl.ds(wid * 128, 128)])

    return sck(symp_tab, dise_tab, idxA, idxB, idxC, idxD, idxE)


def _wfn(cnt):
    w = 1.0 / (cnt + 1e-8)
    return jnp.where(w == 1e8, 0.0, w)


def _leaky(x):
    return jnp.where(x > 0, x, 0.2 * x)


def _tc_body(sumA_ref, sumB_ref, embs_ref, embu1_ref, tgt_ref,
             dsd1_ref, dsd2_ref, usu1_ref, usu2_ref, usu3_ref,
             W21_ref, W22_ref, W11_ref, W12_ref,
             Wu3_ref, Wu21_ref, Wu22_ref, Wu1_ref, out_ref):
    blk = 128
    dot = functools.partial(jnp.dot, preferred_element_type=F32)
    W21, W22 = W21_ref[...], W22_ref[...]
    W11, W12 = W11_ref[...], W12_ref[...]
    Wu3, Wu21, Wu22, Wu1 = Wu3_ref[...], Wu21_ref[...], Wu22_ref[...], Wu1_ref[...]

    cnt2 = jnp.sum((dsd2_ref[...] != 0).astype(F32), axis=-1)
    meand = sumA_ref[...] * _wfn(cnt2)[..., None]
    embs = embs_ref[...]
    X = (meand + embs).reshape(20 * blk, D)
    Y = (meand * embs).reshape(20 * blk, D)
    emb_s_1 = _leaky(dot(X, W21) + dot(Y, W22)).reshape(20, blk, D)
    S1 = jnp.sum(emb_s_1, axis=0)
    cnt1 = jnp.sum((dsd1_ref[...] != 0).astype(F32), axis=-1)
    sbar = S1 * _wfn(cnt1)[:, None]
    tgt = tgt_ref[...]
    emb_dise = _leaky(dot(tgt + sbar, W11) + dot(sbar * tgt, W12))

    cnt3 = jnp.sum((usu3_ref[...] != 0).astype(F32), axis=-1)
    meanu3 = sumB_ref[...] * _wfn(cnt3)[..., None]
    emb_u2 = _leaky(dot(meanu3.reshape(25 * blk, D), Wu3)).reshape(5, 5, blk, D)
    S2 = jnp.sum(emb_u2, axis=1)
    cntu2 = jnp.sum((usu2_ref[...] != 0).astype(F32), axis=-1)
    mbar = S2 * _wfn(cntu2)[..., None]
    embu1 = embu1_ref[...]
    Z = _leaky(dot((embu1 + mbar).reshape(5 * blk, D), Wu21)
               + dot((mbar * embu1).reshape(5 * blk, D), Wu22)).reshape(5, blk, D)
    S3 = jnp.sum(Z, axis=0)
    cntu1 = jnp.sum((usu1_ref[...] != 0).astype(F32), axis=-1)
    ubar = S3 * _wfn(cntu1)[:, None]
    emb_user = _leaky(dot(ubar, Wu1))

    pred = jnp.sum(emb_dise * emb_user, axis=1)
    out_ref[...] = pred.reshape(1, 1, blk)


def _tc_stage(sumA, sumB, embs, embu1, tgt, dsd_1, dsd_2, usu_1, usu_2, usu_3,
              W21, W22, W11, W12, Wu3, Wu21, Wu22, Wu1):
    blk = 128
    g = B // blk
    i3 = lambda i: (0, i, 0)
    i2 = lambda i: (i, 0)
    w2 = lambda i: (0, 0)
    in_specs = [
        pl.BlockSpec((20, blk, D), i3),
        pl.BlockSpec((25, blk, D), i3),
        pl.BlockSpec((20, blk, D), i3),
        pl.BlockSpec((5, blk, D), i3),
        pl.BlockSpec((blk, D), i2),
        pl.BlockSpec((blk, 20), i2),
        pl.BlockSpec((20, blk, 20), i3),
        pl.BlockSpec((blk, 5), i2),
        pl.BlockSpec((5, blk, 5), i3),
        pl.BlockSpec((25, blk, 20), i3),
    ] + [pl.BlockSpec((D, D), w2)] * 8
    out = pl.pallas_call(
        _tc_body,
        grid=(g,),
        in_specs=in_specs,
        out_specs=pl.BlockSpec((1, 1, blk), lambda i: (i, 0, 0)),
        out_shape=jax.ShapeDtypeStruct((g, 1, blk), F32),
    )(sumA, sumB, embs, embu1, tgt, dsd_1, dsd_2, usu_1, usu_2, usu_3,
      W21, W22, W11, W12, Wu3, Wu21, Wu22, Wu1)
    return out.reshape(B)


def kernel(symp_tab, dise_tab, W_dsd_2_1, W_dsd_2_2, W_dsd_1_1, W_dsd_1_2,
           W_usu_3, W_usu_2_1, W_usu_2_2, W_usu_1,
           label, dsd_1, dsd_2, usu_1, usu_2, usu_3):
    dsd_1 = dsd_1.astype(I32)
    dsd_2 = dsd_2.astype(I32)
    usu_1 = usu_1.astype(I32)
    usu_2 = usu_2.astype(I32)
    usu_3 = usu_3.astype(I32)
    label = label.astype(I32)

    idxA = dsd_2.reshape(A_GROUPS * 20 // 128, 128)
    idxB = usu_3.reshape(B_GROUPS * 20 // 128, 128)
    idxC = dsd_1.T.reshape(20 * B // 128, 128)
    idxD = usu_1.T.reshape(5 * B // 128, 128)
    idxE = label.reshape(B // 128, 128)

    sumA, sumB, embs, embu1, tgt = _sc_stage(
        symp_tab.astype(F32), dise_tab.astype(F32), idxA, idxB, idxC, idxD, idxE)

    return _tc_stage(
        sumA.reshape(20, B, D), sumB.reshape(25, B, D),
        embs.reshape(20, B, D), embu1.reshape(5, B, D), tgt,
        dsd_1, dsd_2, usu_1, usu_2, usu_3,
        W_dsd_2_1, W_dsd_2_2, W_dsd_1_1, W_dsd_1_2,
        W_usu_3, W_usu_2_1, W_usu_2_2, W_usu_1)
